# Initial kernel scaffold; baseline (speedup 1.0000x reference)
import jax
import jax.numpy as jnp
from jax.experimental import pallas as pl


def kernel(x, adj, edges, struct_w, struct_b, bn_gamma, bn_beta, w0, b0, w1, b1, w2, b2):
    raise NotImplementedError("write your pallas kernel here")


# speedup vs baseline: 2.1985x; 2.1985x over previous
import functools

import jax
import jax.numpy as jnp
import numpy as np
from jax import lax
from jax.experimental import pallas as pl
from jax.experimental.pallas import tpu as pltpu
from jax.experimental.pallas import tpu_sc as plsc

N = 4096
E = 65536
B = 1024

SLAB_AROWS = 256
NUM_SLABS = N // SLAB_AROWS
SLAB_ELEMS = SLAB_AROWS * N
JUNK_ELEMS = 4096
EDGES_PER_TILE = E // 16
ZCHUNK = (SLAB_ELEMS + JUNK_ELEMS) // 16 // 4


def _scatter_body(edges_hbm, a_out, src_v, dst_v, slab, idxb, ones_v, zbuf,
                  sem):
    c = lax.axis_index("c")
    s = lax.axis_index("s")
    iota16 = lax.iota(jnp.int32, 16)
    ones16 = jnp.ones((16,), jnp.float32)
    zeros16 = jnp.zeros((16,), jnp.float32)

    ebase = s * EDGES_PER_TILE
    pltpu.sync_copy(edges_hbm.at[0, pl.ds(ebase, EDGES_PER_TILE)], src_v)
    pltpu.sync_copy(edges_hbm.at[1, pl.ds(ebase, EDGES_PER_TILE)], dst_v)

    def _z1(i, _):
        zbuf[pl.ds(i * 16, 16)] = zeros16
        return 0
    lax.fori_loop(0, ZCHUNK // 16, _z1, 0)

    for i in range(8):
        ones_v[pl.ds(i * 16, 16)] = ones16

    def _slab_step(k, _):
        slab_id = c * (NUM_SLABS // 2) + k
        row_lo = slab_id * SLAB_AROWS
        elem_base = slab_id * SLAB_ELEMS

        for j in range(4):
            pltpu.sync_copy(
                zbuf, slab.at[pl.ds((s * 4 + j) * ZCHUNK, ZCHUNK)])
        plsc.subcore_barrier()

        def _batch(b, _):
            for t in range(4):
                i = b * 4 + t
                sv = src_v[pl.ds(i * 16, 16)]
                dv = dst_v[pl.ds(i * 16, 16)]
                neq = ((sv - dv) | (dv - sv)) >> 31
                for which in range(2):
                    rs = sv if which == 0 else dv
                    rd = dv if which == 0 else sv
                    pos = t * 32 + which * 16
                    oor = ((rs - row_lo)
                           | (row_lo + SLAB_AROWS - 1 - rs)) >> 31
                    okm = (~oor) & neq
                    junk = SLAB_ELEMS + ((iota16 * 8 + i * 32 + pos)
                                         & (JUNK_ELEMS - 1))
                    idx = ((rs * N + rd - elem_base) & okm) | (junk & (~okm))
                    idxb[0, pl.ds(pos, 16)] = idx
            pltpu.sync_copy(ones_v, slab.at[idxb.at[0]], add=True)
            return 0

        lax.fori_loop(0, EDGES_PER_TILE // 64, _batch, 0)
        plsc.subcore_barrier()

        pltpu.sync_copy(
            slab.at[pl.ds(s * (SLAB_ELEMS // 16), SLAB_ELEMS // 16)],
            a_out.at[pl.ds(elem_base + s * (SLAB_ELEMS // 16),
                           SLAB_ELEMS // 16)])
        plsc.subcore_barrier()
        return 0

    lax.fori_loop(0, NUM_SLABS // 2, _slab_step, 0)


def _build_a(adj):
    mesh = plsc.VectorSubcoreMesh(core_axis_name="c", subcore_axis_name="s")
    k = pl.kernel(
        _scatter_body,
        out_type=jax.ShapeDtypeStruct((N * N,), jnp.float32),
        mesh=mesh,
        scratch_types=[
            pltpu.VMEM((EDGES_PER_TILE,), jnp.int32),
            pltpu.VMEM((EDGES_PER_TILE,), jnp.int32),
            pltpu.VMEM_SHARED((SLAB_ELEMS + JUNK_ELEMS,), jnp.float32),
            pltpu.VMEM((1, 128), jnp.int32),
            pltpu.VMEM((128,), jnp.float32),
            pltpu.VMEM((ZCHUNK,), jnp.float32),
            pltpu.SemaphoreType.DMA,
        ],
    )
    return k(adj).reshape(N, N)


def _gather_body(a_hbm, x_hbm, uv_hbm, d_out, xu_out, xv_out, idx_v, rows_a,
                 rows_x, sem):
    c = lax.axis_index("c")
    s = lax.axis_index("s")
    wid = s * 2 + c
    base = wid * 64
    pltpu.sync_copy(uv_hbm.at[pl.ds(base, 64)], idx_v)
    for j in range(4):
        pltpu.async_copy(a_hbm.at[idx_v.at[pl.ds(j * 16, 16)]], rows_a,
                         sem).wait()
        pltpu.sync_copy(rows_a, d_out.at[pl.ds(base + j * 16, 16)])
    xbase = wid * 32
    pltpu.sync_copy(uv_hbm.at[pl.ds(xbase, 32)], idx_v.at[pl.ds(0, 32)])
    pltpu.async_copy(x_hbm.at[idx_v.at[pl.ds(0, 32)]], rows_x, sem).wait()
    pltpu.sync_copy(rows_x, xu_out.at[pl.ds(xbase, 32)])
    pltpu.sync_copy(uv_hbm.at[pl.ds(B + xbase, 32)], idx_v.at[pl.ds(0, 32)])
    pltpu.async_copy(x_hbm.at[idx_v.at[pl.ds(0, 32)]], rows_x, sem).wait()
    pltpu.sync_copy(rows_x, xv_out.at[pl.ds(xbase, 32)])


def _gather(a, x, uv):
    mesh = plsc.VectorSubcoreMesh(core_axis_name="c", subcore_axis_name="s")
    k = pl.kernel(
        _gather_body,
        out_type=(
            jax.ShapeDtypeStruct((2 * B, N), jnp.float32),
            jax.ShapeDtypeStruct((B, 128), jnp.float32),
            jax.ShapeDtypeStruct((B, 128), jnp.float32),
        ),
        mesh=mesh,
        scratch_types=[
            pltpu.VMEM((64,), jnp.int32),
            pltpu.VMEM((16, N), jnp.float32),
            pltpu.VMEM((32, 128), jnp.float32),
            pltpu.SemaphoreType.DMA,
        ],
    )
    return k(a, x, uv)


BM, BN, BK = 512, 512, 512


def _mm_body(d_ref, a_ref, o_ref, acc):
    @pl.when(pl.program_id(2) == 0)
    def _():
        acc[...] = jnp.zeros_like(acc)

    acc[...] += jnp.dot(
        d_ref[...].astype(jnp.bfloat16),
        a_ref[...].astype(jnp.bfloat16),
        preferred_element_type=jnp.float32)

    @pl.when(pl.program_id(2) == pl.num_programs(2) - 1)
    def _():
        o_ref[...] = acc[...]


def _matmul(d, a):
    return pl.pallas_call(
        _mm_body,
        grid=(2 * B // BM, N // BN, N // BK),
        in_specs=[
            pl.BlockSpec((BM, BK), lambda i, j, k: (i, k)),
            pl.BlockSpec((BK, BN), lambda i, j, k: (k, j)),
        ],
        out_specs=pl.BlockSpec((BM, BN), lambda i, j, k: (i, j)),
        out_shape=jax.ShapeDtypeStruct((2 * B, N), jnp.float32),
        scratch_shapes=[pltpu.VMEM((BM, BN), jnp.float32)],
        compiler_params=pltpu.CompilerParams(
            dimension_semantics=("parallel", "parallel", "arbitrary")),
    )(d, a)


CB = 512


def _cnt_body(du_ref, dv_ref, ru_ref, rv_ref, ub_ref, vb_ref, xu_ref, xv_ref,
              swt_ref, sb_ref, bns_ref, bnb_ref, w0x_ref, w0h_ref, b0_ref,
              w1t_ref, b1_ref, w2p_ref, b2p_ref, o_ref, cacc):
    j = pl.program_id(0)

    @pl.when(j == 0)
    def _():
        cacc[...] = jnp.zeros_like(cacc)

    cols = j * CB + lax.broadcasted_iota(jnp.int32, (B, CB), 1)
    uu = ub_ref[...][:, :1]
    vv = vb_ref[...][:, :1]
    eu = cols == uu
    ev = cols == vv
    valid = jnp.logical_and(~eu, ~ev).astype(jnp.float32)
    bu = du_ref[...] > 0
    bv = dv_ref[...] > 0
    ru = ru_ref[...] > 0
    rv = rv_ref[...] > 0
    d2u = ru & ~bu & ~eu
    d2v = rv & ~bv & ~ev
    infu = ~bu & ~d2u & ~eu
    infv = ~bv & ~d2v & ~ev

    f32 = lambda m: m.astype(jnp.float32)
    t11 = f32(bu & bv)
    t12 = f32(bu & d2v) + f32(d2u & bv)
    t1i = f32(bu & infv) + f32(infu & bv)
    t22 = f32(d2u & d2v)
    t2i = f32(d2u & infv) + f32(infu & d2v)

    def rs(t):
        return jnp.sum(t * valid, axis=1, keepdims=True)

    part = jnp.concatenate(
        [rs(t11), rs(t12), rs(t1i), rs(t22), rs(t2i),
         jnp.zeros((B, 3), jnp.float32)], axis=1)
    cacc[...] += part

    @pl.when(j == pl.num_programs(0) - 1)
    def _():
        cnt = cacc[...]
        h = jnp.dot(cnt, swt_ref[...], preferred_element_type=jnp.float32)
        h = (h + sb_ref[...]) * bns_ref[...] + bnb_ref[...]
        h = jnp.maximum(h, 0.0)
        xuv = xu_ref[...] * xv_ref[...]
        z = jnp.dot(xuv, w0x_ref[...], preferred_element_type=jnp.float32)
        z += jnp.dot(h, w0h_ref[...], preferred_element_type=jnp.float32)
        z = jnp.maximum(z + b0_ref[...], 0.0)
        z = jnp.dot(z, w1t_ref[...], preferred_element_type=jnp.float32)
        z = jnp.maximum(z + b1_ref[...], 0.0)
        o = jnp.dot(z, w2p_ref[...], preferred_element_type=jnp.float32)
        o_ref[...] = o + b2p_ref[...]


def _counts_mlp(d, r2, ub, vb, xu, xv, swt8, sb8, bns8, bnb8, w0xt, w0ht8,
                b0r, w1t, b1r, w2p, b2p):
    full = lambda shape: pl.BlockSpec(shape, lambda j: (0,) * len(shape))
    return pl.pallas_call(
        _cnt_body,
        grid=(N // CB,),
        in_specs=[
            pl.BlockSpec((B, CB), lambda j: (0, j)),
            pl.BlockSpec((B, CB), lambda j: (1, j)),
            pl.BlockSpec((B, CB), lambda j: (0, j)),
            pl.BlockSpec((B, CB), lambda j: (1, j)),
            full((B, 128)),
            full((B, 128)),
            full((B, 128)),
            full((B, 128)),
            full((8, 8)),
            full((1, 8)),
            full((1, 8)),
            full((1, 8)),
            full((128, 256)),
            full((8, 256)),
            full((1, 256)),
            full((256, 256)),
            full((1, 256)),
            full((256, 128)),
            full((1, 128)),
        ],
        out_specs=full((B, 128)),
        out_shape=jax.ShapeDtypeStruct((B, 128), jnp.float32),
        scratch_shapes=[pltpu.VMEM((B, 8), jnp.float32)],
        compiler_params=pltpu.CompilerParams(
            dimension_semantics=("arbitrary",)),
    )(d, d, r2, r2, ub, vb, xu, xv, swt8, sb8, bns8, bnb8, w0xt, w0ht8, b0r,
      w1t, b1r, w2p, b2p)


def kernel(x, adj, edges, struct_w, struct_b, bn_gamma, bn_beta, w0, b0, w1,
           b1, w2, b2):
    a = _build_a(adj)
    u = edges[0]
    v = edges[1]
    uv = jnp.concatenate([u, v])
    d, xu, xv = _gather(a, x, uv)
    r2 = _matmul(d, a)

    ub = jnp.broadcast_to(u[:, None], (B, 128))
    vb = jnp.broadcast_to(v[:, None], (B, 128))
    swt8 = jnp.zeros((8, 8), jnp.float32).at[:5, :5].set(struct_w.T)
    sb8 = jnp.zeros((1, 8), jnp.float32).at[0, :5].set(struct_b)
    bns8 = jnp.zeros((1, 8), jnp.float32).at[0, :5].set(
        bn_gamma / np.sqrt(1.0 + 1e-05))
    bnb8 = jnp.zeros((1, 8), jnp.float32).at[0, :5].set(bn_beta)
    w0xt = w0[:, :128].T
    w0ht8 = jnp.zeros((8, 256), jnp.float32).at[:5].set(w0[:, 128:].T)
    b0r = b0[None, :]
    w1t = w1.T
    b1r = b1[None, :]
    w2p = jnp.zeros((256, 128), jnp.float32).at[:, 0].set(w2[0])
    b2p = jnp.zeros((1, 128), jnp.float32).at[0, 0].set(b2[0])

    out = _counts_mlp(d, r2, ub, vb, xu, xv, swt8, sb8, bns8, bnb8, w0xt,
                      w0ht8, b0r, w1t, b1r, w2p, b2p)
    return out[:, :1]


# speedup vs baseline: 2.3445x; 1.0664x over previous
import jax
import jax.numpy as jnp
import numpy as np
from jax import lax
from jax.experimental import pallas as pl
from jax.experimental.pallas import tpu as pltpu
from jax.experimental.pallas import tpu_sc as plsc

N = 4096
E = 65536
B = 1024

EDGES_PER_TILE = E // 16
HALF = N * N // 2
ZCH = 16384


def _scatter_body(edges_hbm, a_out, src_v, dst_v, fireb, ones_v, zbuf, sem):
    c = lax.axis_index("c")
    s = lax.axis_index("s")
    zeros16 = jnp.zeros((16,), jnp.float32)
    ones16 = jnp.ones((16,), jnp.float32)

    ebase = s * EDGES_PER_TILE
    pltpu.sync_copy(edges_hbm.at[0, pl.ds(ebase, EDGES_PER_TILE)], src_v)
    pltpu.sync_copy(edges_hbm.at[1, pl.ds(ebase, EDGES_PER_TILE)], dst_v)

    def _z1(i, _):
        zbuf[pl.ds(i * 16, 16)] = zeros16
        return 0
    lax.fori_loop(0, ZCH // 16, _z1, 0)
    for i in range(8):
        ones_v[pl.ds(i * 16, 16)] = ones16

    tbase = c * HALF + s * (HALF // 16)
    for q in range(HALF // 16 // ZCH):
        pltpu.sync_copy(zbuf, a_out.at[pl.ds(tbase + q * ZCH, ZCH)])
    plsc.subcore_barrier()

    iota16 = lax.iota(jnp.int32, 16)

    def _batch(b, _):
        @pl.when(b >= 2)
        def _():
            pltpu.make_async_copy(ones_v, a_out.at[fireb.at[0]], sem).wait()

        def _build(slot):
            for t in range(4):
                i = b * 4 + t
                sv = src_v[pl.ds(i * 16, 16)]
                dv = dst_v[pl.ds(i * 16, 16)]
                for which in range(2):
                    rs = sv if which == 0 else dv
                    rd = dv if which == 0 else sv
                    okm = ((rs >> 11) ^ c) - 1
                    trash = (c * (N // 2) + s * 128
                             + ((iota16 * 8 + i) & 127)) * (N + 1)
                    idx = ((rs * N + rd) & okm) | (trash & (~okm))
                    fireb[slot, pl.ds(t * 32 + which * 16, 16)] = idx
            pltpu.async_copy(ones_v, a_out.at[fireb.at[slot]], sem)

        @pl.when(b % 2 == 0)
        def _():
            _build(0)

        @pl.when(b % 2 == 1)
        def _():
            _build(1)
        return 0

    lax.fori_loop(0, 2 * EDGES_PER_TILE // 128, _batch, 0)
    pltpu.make_async_copy(ones_v, a_out.at[fireb.at[0]], sem).wait()
    pltpu.make_async_copy(ones_v, a_out.at[fireb.at[0]], sem).wait()


def _build_a(adj):
    mesh = plsc.VectorSubcoreMesh(core_axis_name="c", subcore_axis_name="s")
    k = pl.kernel(
        _scatter_body,
        out_type=jax.ShapeDtypeStruct((N * N,), jnp.float32),
        mesh=mesh,
        scratch_types=[
            pltpu.VMEM((EDGES_PER_TILE,), jnp.int32),
            pltpu.VMEM((EDGES_PER_TILE,), jnp.int32),
            pltpu.VMEM((2, 128), jnp.int32),
            pltpu.VMEM((128,), jnp.float32),
            pltpu.VMEM((ZCH,), jnp.float32),
            pltpu.SemaphoreType.DMA,
        ],
    )
    return k(adj).reshape(N, N)


def _gather_body(a_hbm, x_hbm, uv_hbm, d_out, xu_out, xv_out, idx_v, rows_a,
                 rows_x, sem):
    c = lax.axis_index("c")
    s = lax.axis_index("s")
    wid = s * 2 + c
    base = wid * 64
    pltpu.sync_copy(uv_hbm.at[pl.ds(base, 64)], idx_v)
    for j in range(4):
        pltpu.async_copy(a_hbm.at[idx_v.at[pl.ds(j * 16, 16)]], rows_a,
                         sem).wait()
        pltpu.sync_copy(rows_a, d_out.at[pl.ds(base + j * 16, 16)])
    xbase = wid * 32
    pltpu.sync_copy(uv_hbm.at[pl.ds(xbase, 32)], idx_v.at[pl.ds(0, 32)])
    pltpu.async_copy(x_hbm.at[idx_v.at[pl.ds(0, 32)]], rows_x, sem).wait()
    pltpu.sync_copy(rows_x, xu_out.at[pl.ds(xbase, 32)])
    pltpu.sync_copy(uv_hbm.at[pl.ds(B + xbase, 32)], idx_v.at[pl.ds(0, 32)])
    pltpu.async_copy(x_hbm.at[idx_v.at[pl.ds(0, 32)]], rows_x, sem).wait()
    pltpu.sync_copy(rows_x, xv_out.at[pl.ds(xbase, 32)])


def _gather(a, x, uv):
    mesh = plsc.VectorSubcoreMesh(core_axis_name="c", subcore_axis_name="s")
    k = pl.kernel(
        _gather_body,
        out_type=(
            jax.ShapeDtypeStruct((2 * B, N), jnp.float32),
            jax.ShapeDtypeStruct((B, 128), jnp.float32),
            jax.ShapeDtypeStruct((B, 128), jnp.float32),
        ),
        mesh=mesh,
        scratch_types=[
            pltpu.VMEM((64,), jnp.int32),
            pltpu.VMEM((16, N), jnp.float32),
            pltpu.VMEM((32, 128), jnp.float32),
            pltpu.SemaphoreType.DMA,
        ],
    )
    return k(a, x, uv)


CB = 512
KB = 512


def _fused_body(d_ref, a_ref, dc_ref, ub_ref, vb_ref, xu_ref, xv_ref,
                swt_ref, sb_ref, bns_ref, bnb_ref, w0x_ref, w0h_ref, b0_ref,
                w1t_ref, b1_ref, w2p_ref, b2p_ref, o_ref, cacc):
    j = pl.program_id(0)

    @pl.when(j == 0)
    def _():
        cacc[...] = jnp.zeros_like(cacc)

    acc = jnp.zeros((2 * B, CB), jnp.float32)
    for k in range(N // KB):
        acc += jnp.dot(
            d_ref[:, k * KB:(k + 1) * KB],
            a_ref[k * KB:(k + 1) * KB, :].astype(jnp.bfloat16),
            preferred_element_type=jnp.float32)

    cols = j * CB + lax.broadcasted_iota(jnp.int32, (B, CB), 1)
    uu = ub_ref[...][:, :1]
    vv = vb_ref[...][:, :1]
    eu = cols == uu
    ev = cols == vv
    valid = jnp.logical_and(~eu, ~ev).astype(jnp.float32)
    bu = dc_ref[:B, :] > 0
    bv = dc_ref[B:, :] > 0
    ru = acc[:B, :] > 0
    rv = acc[B:, :] > 0
    d2u = ru & ~bu & ~eu
    d2v = rv & ~bv & ~ev
    infu = ~bu & ~d2u & ~eu
    infv = ~bv & ~d2v & ~ev

    f32 = lambda m: m.astype(jnp.float32)
    t11 = f32(bu & bv)
    t12 = f32(bu & d2v) + f32(d2u & bv)
    t1i = f32(bu & infv) + f32(infu & bv)
    t22 = f32(d2u & d2v)
    t2i = f32(d2u & infv) + f32(infu & d2v)

    def rs(t):
        return jnp.sum(t * valid, axis=1, keepdims=True)

    part = jnp.concatenate(
        [rs(t11), rs(t12), rs(t1i), rs(t22), rs(t2i),
         jnp.zeros((B, 3), jnp.float32)], axis=1)
    cacc[...] += part

    @pl.when(j == pl.num_programs(0) - 1)
    def _():
        cnt = cacc[...]
        h = jnp.dot(cnt, swt_ref[...], preferred_element_type=jnp.float32)
        h = (h + sb_ref[...]) * bns_ref[...] + bnb_ref[...]
        h = jnp.maximum(h, 0.0)
        xuv = xu_ref[...] * xv_ref[...]
        z = jnp.dot(xuv, w0x_ref[...], preferred_element_type=jnp.float32)
        z += jnp.dot(h, w0h_ref[...], preferred_element_type=jnp.float32)
        z = jnp.maximum(z + b0_ref[...], 0.0)
        z = jnp.dot(z, w1t_ref[...], preferred_element_type=jnp.float32)
        z = jnp.maximum(z + b1_ref[...], 0.0)
        o = jnp.dot(z, w2p_ref[...], preferred_element_type=jnp.float32)
        o_ref[...] = o + b2p_ref[...]


def _fused(d, a, ub, vb, xu, xv, swt8, sb8, bns8, bnb8, w0xt, w0ht8, b0r,
           w1t, b1r, w2p, b2p):
    full = lambda shape: pl.BlockSpec(shape, lambda j: (0,) * len(shape))
    return pl.pallas_call(
        _fused_body,
        grid=(N // CB,),
        in_specs=[
            full((2 * B, N)),
            pl.BlockSpec((N, CB), lambda j: (0, j)),
            pl.BlockSpec((2 * B, CB), lambda j: (0, j)),
            full((B, 128)),
            full((B, 128)),
            full((B, 128)),
            full((B, 128)),
            full((8, 8)),
            full((1, 8)),
            full((1, 8)),
            full((1, 8)),
            full((128, 256)),
            full((8, 256)),
            full((1, 256)),
            full((256, 256)),
            full((1, 256)),
            full((256, 128)),
            full((1, 128)),
        ],
        out_specs=full((B, 128)),
        out_shape=jax.ShapeDtypeStruct((B, 128), jnp.float32),
        scratch_shapes=[pltpu.VMEM((B, 8), jnp.float32)],
        compiler_params=pltpu.CompilerParams(
            dimension_semantics=("arbitrary",)),
    )(d, a, d, ub, vb, xu, xv, swt8, sb8, bns8, bnb8, w0xt, w0ht8, b0r, w1t,
      b1r, w2p, b2p)


def kernel(x, adj, edges, struct_w, struct_b, bn_gamma, bn_beta, w0, b0, w1,
           b1, w2, b2):
    a = _build_a(adj)
    u = edges[0]
    v = edges[1]
    uv = jnp.concatenate([u, v])
    d, xu, xv = _gather(a, x, uv)
    d = d.astype(jnp.bfloat16)

    ub = jnp.broadcast_to(u[:, None], (B, 128))
    vb = jnp.broadcast_to(v[:, None], (B, 128))
    swt8 = jnp.zeros((8, 8), jnp.float32).at[:5, :5].set(struct_w.T)
    sb8 = jnp.zeros((1, 8), jnp.float32).at[0, :5].set(struct_b)
    bns8 = jnp.zeros((1, 8), jnp.float32).at[0, :5].set(
        bn_gamma / np.sqrt(1.0 + 1e-05))
    bnb8 = jnp.zeros((1, 8), jnp.float32).at[0, :5].set(bn_beta)
    w0xt = w0[:, :128].T
    w0ht8 = jnp.zeros((8, 256), jnp.float32).at[:5].set(w0[:, 128:].T)
    b0r = b0[None, :]
    w1t = w1.T
    b1r = b1[None, :]
    w2p = jnp.zeros((256, 128), jnp.float32).at[:, 0].set(w2[0])
    b2p = jnp.zeros((1, 128), jnp.float32).at[0, 0].set(b2[0])

    out = _fused(d, a, ub, vb, xu, xv, swt8, sb8, bns8, bnb8, w0xt, w0ht8,
                 b0r, w1t, b1r, w2p, b2p)
    return out[:, :1]


# speedup vs baseline: 3.4245x; 1.4607x over previous
import jax
import jax.numpy as jnp
import numpy as np
from jax import lax
from jax.experimental import pallas as pl
from jax.experimental.pallas import tpu as pltpu
from jax.experimental.pallas import tpu_sc as plsc

N = 4096
E = 65536
B = 1024

SLAB_AROWS = 256
NUM_SLABS = N // SLAB_AROWS
SLAB_ELEMS = SLAB_AROWS * N
JUNK_ELEMS = 4096
EDGES_PER_TILE = E // 16
ZCHUNK = (SLAB_ELEMS + JUNK_ELEMS) // 16 // 4


def _scatter_body(edges_hbm, a_out, src_v, dst_v, slab, idxb, ones_v, zbuf,
                  sem):
    c = lax.axis_index("c")
    s = lax.axis_index("s")
    iota16 = lax.iota(jnp.int32, 16)
    ones16 = jnp.ones((16,), jnp.float32)
    zeros16 = jnp.zeros((16,), jnp.float32)

    ebase = s * EDGES_PER_TILE
    pltpu.sync_copy(edges_hbm.at[0, pl.ds(ebase, EDGES_PER_TILE)], src_v)
    pltpu.sync_copy(edges_hbm.at[1, pl.ds(ebase, EDGES_PER_TILE)], dst_v)

    def _z1(i, _):
        zbuf[pl.ds(i * 16, 16)] = zeros16
        return 0
    lax.fori_loop(0, ZCHUNK // 16, _z1, 0)

    for i in range(8):
        ones_v[pl.ds(i * 16, 16)] = ones16

    def _slab_step(k, _):
        slab_id = c * (NUM_SLABS // 2) + k
        row_lo = slab_id * SLAB_AROWS
        elem_base = slab_id * SLAB_ELEMS

        for j in range(4):
            pltpu.sync_copy(
                zbuf, slab.at[pl.ds((s * 4 + j) * ZCHUNK, ZCHUNK)])
        plsc.subcore_barrier()

        def _batch(b, _):
            @pl.when(b >= 2)
            def _():
                pltpu.make_async_copy(ones_v, slab.at[idxb.at[0]],
                                      sem).wait()

            def _build(slot):
                for t in range(4):
                    i = b * 4 + t
                    sv = src_v[pl.ds(i * 16, 16)]
                    dv = dst_v[pl.ds(i * 16, 16)]
                    neq = ((sv - dv) | (dv - sv)) >> 31
                    for which in range(2):
                        rs = sv if which == 0 else dv
                        rd = dv if which == 0 else sv
                        pos = t * 32 + which * 16
                        oor = ((rs - row_lo)
                               | (row_lo + SLAB_AROWS - 1 - rs)) >> 31
                        okm = (~oor) & neq
                        junk = SLAB_ELEMS + ((iota16 * 8 + i * 32 + pos)
                                             & (JUNK_ELEMS - 1))
                        idx = (((rs * N + rd - elem_base) & okm)
                               | (junk & (~okm)))
                        idxb[slot, pl.ds(pos, 16)] = idx
                pltpu.async_copy(ones_v, slab.at[idxb.at[slot]], sem,
                                 add=True)

            @pl.when(b % 2 == 0)
            def _():
                _build(0)

            @pl.when(b % 2 == 1)
            def _():
                _build(1)
            return 0

        lax.fori_loop(0, EDGES_PER_TILE // 64, _batch, 0)
        pltpu.make_async_copy(ones_v, slab.at[idxb.at[0]], sem).wait()
        pltpu.make_async_copy(ones_v, slab.at[idxb.at[0]], sem).wait()
        plsc.subcore_barrier()

        pltpu.sync_copy(
            slab.at[pl.ds(s * (SLAB_ELEMS // 16), SLAB_ELEMS // 16)],
            a_out.at[pl.ds(elem_base + s * (SLAB_ELEMS // 16),
                           SLAB_ELEMS // 16)])
        plsc.subcore_barrier()
        return 0

    lax.fori_loop(0, NUM_SLABS // 2, _slab_step, 0)


def _build_a(adj):
    mesh = plsc.VectorSubcoreMesh(core_axis_name="c", subcore_axis_name="s")
    k = pl.kernel(
        _scatter_body,
        out_type=jax.ShapeDtypeStruct((N * N,), jnp.float32),
        mesh=mesh,
        scratch_types=[
            pltpu.VMEM((EDGES_PER_TILE,), jnp.int32),
            pltpu.VMEM((EDGES_PER_TILE,), jnp.int32),
            pltpu.VMEM_SHARED((SLAB_ELEMS + JUNK_ELEMS,), jnp.float32),
            pltpu.VMEM((2, 128), jnp.int32),
            pltpu.VMEM((128,), jnp.float32),
            pltpu.VMEM((ZCHUNK,), jnp.float32),
            pltpu.SemaphoreType.DMA,
        ],
    )
    return k(adj).reshape(N, N)


def _gather_body(a_hbm, x_hbm, uv_hbm, d_out, xu_out, xv_out, idx_v, rows_a,
                 rows_x, sem):
    c = lax.axis_index("c")
    s = lax.axis_index("s")
    wid = s * 2 + c
    base = wid * 64
    pltpu.sync_copy(uv_hbm.at[pl.ds(base, 64)], idx_v)
    for j in range(4):
        pltpu.async_copy(a_hbm.at[idx_v.at[pl.ds(j * 16, 16)]], rows_a,
                         sem).wait()
        pltpu.sync_copy(rows_a, d_out.at[pl.ds(base + j * 16, 16)])
    xbase = wid * 32
    pltpu.sync_copy(uv_hbm.at[pl.ds(xbase, 32)], idx_v.at[pl.ds(0, 32)])
    pltpu.async_copy(x_hbm.at[idx_v.at[pl.ds(0, 32)]], rows_x, sem).wait()
    pltpu.sync_copy(rows_x, xu_out.at[pl.ds(xbase, 32)])
    pltpu.sync_copy(uv_hbm.at[pl.ds(B + xbase, 32)], idx_v.at[pl.ds(0, 32)])
    pltpu.async_copy(x_hbm.at[idx_v.at[pl.ds(0, 32)]], rows_x, sem).wait()
    pltpu.sync_copy(rows_x, xv_out.at[pl.ds(xbase, 32)])


def _gather(a, x, uv):
    mesh = plsc.VectorSubcoreMesh(core_axis_name="c", subcore_axis_name="s")
    k = pl.kernel(
        _gather_body,
        out_type=(
            jax.ShapeDtypeStruct((2 * B, N), jnp.float32),
            jax.ShapeDtypeStruct((B, 128), jnp.float32),
            jax.ShapeDtypeStruct((B, 128), jnp.float32),
        ),
        mesh=mesh,
        scratch_types=[
            pltpu.VMEM((64,), jnp.int32),
            pltpu.VMEM((16, N), jnp.float32),
            pltpu.VMEM((32, 128), jnp.float32),
            pltpu.SemaphoreType.DMA,
        ],
    )
    return k(a, x, uv)


CB = 512
KB = 512


def _fused_body(d_ref, a_ref, dc_ref, ub_ref, vb_ref, xu_ref, xv_ref,
                swt_ref, sb_ref, bns_ref, bnb_ref, w0x_ref, w0h_ref, b0_ref,
                w1t_ref, b1_ref, w2p_ref, b2p_ref, o_ref, cacc):
    j = pl.program_id(0)

    @pl.when(j == 0)
    def _():
        cacc[...] = jnp.zeros_like(cacc)

    acc = jnp.zeros((2 * B, CB), jnp.float32)
    for k in range(N // KB):
        acc += jnp.dot(
            d_ref[:, k * KB:(k + 1) * KB],
            a_ref[k * KB:(k + 1) * KB, :].astype(jnp.bfloat16),
            preferred_element_type=jnp.float32)

    cols = j * CB + lax.broadcasted_iota(jnp.int32, (B, CB), 1)
    uu = ub_ref[...][:, :1]
    vv = vb_ref[...][:, :1]
    eu = cols == uu
    ev = cols == vv
    valid = jnp.logical_and(~eu, ~ev).astype(jnp.float32)
    bu = dc_ref[:B, :] > 0
    bv = dc_ref[B:, :] > 0
    ru = acc[:B, :] > 0
    rv = acc[B:, :] > 0
    d2u = ru & ~bu & ~eu
    d2v = rv & ~bv & ~ev
    infu = ~bu & ~d2u & ~eu
    infv = ~bv & ~d2v & ~ev

    f32 = lambda m: m.astype(jnp.float32)
    t11 = f32(bu & bv)
    t12 = f32(bu & d2v) + f32(d2u & bv)
    t1i = f32(bu & infv) + f32(infu & bv)
    t22 = f32(d2u & d2v)
    t2i = f32(d2u & infv) + f32(infu & d2v)

    def rs(t):
        return jnp.sum(t * valid, axis=1, keepdims=True)

    part = jnp.concatenate(
        [rs(t11), rs(t12), rs(t1i), rs(t22), rs(t2i),
         jnp.zeros((B, 3), jnp.float32)], axis=1)
    cacc[...] += part

    @pl.when(j == pl.num_programs(0) - 1)
    def _():
        cnt = cacc[...]
        h = jnp.dot(cnt, swt_ref[...], preferred_element_type=jnp.float32)
        h = (h + sb_ref[...]) * bns_ref[...] + bnb_ref[...]
        h = jnp.maximum(h, 0.0)
        xuv = xu_ref[...] * xv_ref[...]
        z = jnp.dot(xuv, w0x_ref[...], preferred_element_type=jnp.float32)
        z += jnp.dot(h, w0h_ref[...], preferred_element_type=jnp.float32)
        z = jnp.maximum(z + b0_ref[...], 0.0)
        z = jnp.dot(z, w1t_ref[...], preferred_element_type=jnp.float32)
        z = jnp.maximum(z + b1_ref[...], 0.0)
        o = jnp.dot(z, w2p_ref[...], preferred_element_type=jnp.float32)
        o_ref[...] = o + b2p_ref[...]


def _fused(d, a, ub, vb, xu, xv, swt8, sb8, bns8, bnb8, w0xt, w0ht8, b0r,
           w1t, b1r, w2p, b2p):
    full = lambda shape: pl.BlockSpec(shape, lambda j: (0,) * len(shape))
    return pl.pallas_call(
        _fused_body,
        grid=(N // CB,),
        in_specs=[
            full((2 * B, N)),
            pl.BlockSpec((N, CB), lambda j: (0, j)),
            pl.BlockSpec((2 * B, CB), lambda j: (0, j)),
            full((B, 128)),
            full((B, 128)),
            full((B, 128)),
            full((B, 128)),
            full((8, 8)),
            full((1, 8)),
            full((1, 8)),
            full((1, 8)),
            full((128, 256)),
            full((8, 256)),
            full((1, 256)),
            full((256, 256)),
            full((1, 256)),
            full((256, 128)),
            full((1, 128)),
        ],
        out_specs=full((B, 128)),
        out_shape=jax.ShapeDtypeStruct((B, 128), jnp.float32),
        scratch_shapes=[pltpu.VMEM((B, 8), jnp.float32)],
        compiler_params=pltpu.CompilerParams(
            dimension_semantics=("arbitrary",)),
    )(d, a, d, ub, vb, xu, xv, swt8, sb8, bns8, bnb8, w0xt, w0ht8, b0r, w1t,
      b1r, w2p, b2p)


def kernel(x, adj, edges, struct_w, struct_b, bn_gamma, bn_beta, w0, b0, w1,
           b1, w2, b2):
    a = _build_a(adj)
    u = edges[0]
    v = edges[1]
    uv = jnp.concatenate([u, v])
    d, xu, xv = _gather(a, x, uv)
    d = d.astype(jnp.bfloat16)

    ub = jnp.broadcast_to(u[:, None], (B, 128))
    vb = jnp.broadcast_to(v[:, None], (B, 128))
    swt8 = jnp.zeros((8, 8), jnp.float32).at[:5, :5].set(struct_w.T)
    sb8 = jnp.zeros((1, 8), jnp.float32).at[0, :5].set(struct_b)
    bns8 = jnp.zeros((1, 8), jnp.float32).at[0, :5].set(
        bn_gamma / np.sqrt(1.0 + 1e-05))
    bnb8 = jnp.zeros((1, 8), jnp.float32).at[0, :5].set(bn_beta)
    w0xt = w0[:, :128].T
    w0ht8 = jnp.zeros((8, 256), jnp.float32).at[:5].set(w0[:, 128:].T)
    b0r = b0[None, :]
    w1t = w1.T
    b1r = b1[None, :]
    w2p = jnp.zeros((256, 128), jnp.float32).at[:, 0].set(w2[0])
    b2p = jnp.zeros((1, 128), jnp.float32).at[0, 0].set(b2[0])

    out = _fused(d, a, ub, vb, xu, xv, swt8, sb8, bns8, bnb8, w0xt, w0ht8,
                 b0r, w1t, b1r, w2p, b2p)
    return out[:, :1]


# speedup vs baseline: 3.4382x; 1.0040x over previous
import jax
import jax.numpy as jnp
import numpy as np
from jax import lax
from jax.experimental import pallas as pl
from jax.experimental.pallas import tpu as pltpu
from jax.experimental.pallas import tpu_sc as plsc

N = 4096
E = 65536
B = 1024

SLAB_AROWS = 256
NUM_SLABS = N // SLAB_AROWS
SLAB_ELEMS = SLAB_AROWS * N
JUNK_ELEMS = 4096
EDGES_PER_TILE = E // 16
ZCHUNK = (SLAB_ELEMS + JUNK_ELEMS) // 16 // 4


def _scatter_body(edges_hbm, a_out, src_v, dst_v, slab, idxb, ones_v, zbuf,
                  sem):
    c = lax.axis_index("c")
    s = lax.axis_index("s")
    iota16 = lax.iota(jnp.int32, 16)
    ones16 = jnp.ones((16,), jnp.float32)
    zeros16 = jnp.zeros((16,), jnp.float32)

    ebase = s * EDGES_PER_TILE
    pltpu.sync_copy(edges_hbm.at[0, pl.ds(ebase, EDGES_PER_TILE)], src_v)
    pltpu.sync_copy(edges_hbm.at[1, pl.ds(ebase, EDGES_PER_TILE)], dst_v)

    def _z1(i, _):
        zbuf[pl.ds(i * 16, 16)] = zeros16
        return 0
    lax.fori_loop(0, ZCHUNK // 16, _z1, 0)

    for i in range(8):
        ones_v[pl.ds(i * 16, 16)] = ones16

    def _slab_step(k, _):
        slab_id = c * (NUM_SLABS // 2) + k
        row_lo = slab_id * SLAB_AROWS
        elem_base = slab_id * SLAB_ELEMS

        for j in range(4):
            pltpu.sync_copy(
                zbuf, slab.at[pl.ds((s * 4 + j) * ZCHUNK, ZCHUNK)])
        plsc.subcore_barrier()

        def _batch(b, _):
            @pl.when(b >= 2)
            def _():
                pltpu.make_async_copy(ones_v, slab.at[idxb.at[0]],
                                      sem).wait()

            def _build(slot):
                for t in range(4):
                    i = b * 4 + t
                    sv = src_v[pl.ds(i * 16, 16)]
                    dv = dst_v[pl.ds(i * 16, 16)]
                    neq = ((sv - dv) | (dv - sv)) >> 31
                    for which in range(2):
                        rs = sv if which == 0 else dv
                        rd = dv if which == 0 else sv
                        pos = t * 32 + which * 16
                        oor = ((rs - row_lo)
                               | (row_lo + SLAB_AROWS - 1 - rs)) >> 31
                        okm = (~oor) & neq
                        junk = SLAB_ELEMS + ((iota16 * 8 + i * 32 + pos)
                                             & (JUNK_ELEMS - 1))
                        idx = (((rs * N + rd - elem_base) & okm)
                               | (junk & (~okm)))
                        idxb[slot, pl.ds(pos, 16)] = idx
                pltpu.async_copy(ones_v, slab.at[idxb.at[slot]], sem,
                                 add=True)

            @pl.when(b % 2 == 0)
            def _():
                _build(0)

            @pl.when(b % 2 == 1)
            def _():
                _build(1)
            return 0

        lax.fori_loop(0, EDGES_PER_TILE // 64, _batch, 0)
        pltpu.make_async_copy(ones_v, slab.at[idxb.at[0]], sem).wait()
        pltpu.make_async_copy(ones_v, slab.at[idxb.at[0]], sem).wait()
        plsc.subcore_barrier()

        pltpu.sync_copy(
            slab.at[pl.ds(s * (SLAB_ELEMS // 16), SLAB_ELEMS // 16)],
            a_out.at[pl.ds(elem_base + s * (SLAB_ELEMS // 16),
                           SLAB_ELEMS // 16)])
        plsc.subcore_barrier()
        return 0

    lax.fori_loop(0, NUM_SLABS // 2, _slab_step, 0)


def _build_a(adj):
    mesh = plsc.VectorSubcoreMesh(core_axis_name="c", subcore_axis_name="s")
    k = pl.kernel(
        _scatter_body,
        out_type=jax.ShapeDtypeStruct((N * N,), jnp.float32),
        mesh=mesh,
        scratch_types=[
            pltpu.VMEM((EDGES_PER_TILE,), jnp.int32),
            pltpu.VMEM((EDGES_PER_TILE,), jnp.int32),
            pltpu.VMEM_SHARED((SLAB_ELEMS + JUNK_ELEMS,), jnp.float32),
            pltpu.VMEM((2, 128), jnp.int32),
            pltpu.VMEM((128,), jnp.float32),
            pltpu.VMEM((ZCHUNK,), jnp.float32),
            pltpu.SemaphoreType.DMA,
        ],
    )
    return k(adj).reshape(N, N)


def _gather_body(a_hbm, x_hbm, uv_hbm, d_out, xu_out, xv_out, idx_v, rows_a,
                 rows_x, sem):
    c = lax.axis_index("c")
    s = lax.axis_index("s")
    wid = s * 2 + c
    base = wid * 64
    pltpu.sync_copy(uv_hbm.at[pl.ds(base, 64)], idx_v)
    for j in range(4):
        pltpu.async_copy(a_hbm.at[idx_v.at[pl.ds(j * 16, 16)]], rows_a,
                         sem).wait()
        pltpu.sync_copy(rows_a, d_out.at[pl.ds(base + j * 16, 16)])
    xbase = wid * 32
    pltpu.sync_copy(uv_hbm.at[pl.ds(xbase, 32)], idx_v.at[pl.ds(0, 32)])
    pltpu.async_copy(x_hbm.at[idx_v.at[pl.ds(0, 32)]], rows_x, sem).wait()
    pltpu.sync_copy(rows_x, xu_out.at[pl.ds(xbase, 32)])
    pltpu.sync_copy(uv_hbm.at[pl.ds(B + xbase, 32)], idx_v.at[pl.ds(0, 32)])
    pltpu.async_copy(x_hbm.at[idx_v.at[pl.ds(0, 32)]], rows_x, sem).wait()
    pltpu.sync_copy(rows_x, xv_out.at[pl.ds(xbase, 32)])


def _gather(a, x, uv):
    mesh = plsc.VectorSubcoreMesh(core_axis_name="c", subcore_axis_name="s")
    k = pl.kernel(
        _gather_body,
        out_type=(
            jax.ShapeDtypeStruct((2 * B, N), jnp.float32),
            jax.ShapeDtypeStruct((B, 128), jnp.float32),
            jax.ShapeDtypeStruct((B, 128), jnp.float32),
        ),
        mesh=mesh,
        scratch_types=[
            pltpu.VMEM((64,), jnp.int32),
            pltpu.VMEM((16, N), jnp.float32),
            pltpu.VMEM((32, 128), jnp.float32),
            pltpu.SemaphoreType.DMA,
        ],
    )
    return k(a, x, uv)


CB = 512
KB = 512


def _fused_body(d_ref, a_ref, dc_ref, ub_ref, vb_ref, xu_ref, xv_ref,
                swt_ref, sb_ref, bns_ref, bnb_ref, w0x_ref, w0h_ref, b0_ref,
                w1t_ref, b1_ref, w2p_ref, b2p_ref, o_ref, cacc):
    j = pl.program_id(0)

    @pl.when(j == 0)
    def _():
        cacc[...] = jnp.zeros_like(cacc)

    acc = jnp.zeros((2 * B, CB), jnp.float32)
    for k in range(N // KB):
        acc += jnp.dot(
            d_ref[:, k * KB:(k + 1) * KB],
            a_ref[k * KB:(k + 1) * KB, :].astype(jnp.bfloat16),
            preferred_element_type=jnp.float32)

    cols = j * CB + lax.broadcasted_iota(jnp.int32, (B, CB), 1)
    uu = ub_ref[...][:, :1]
    vv = vb_ref[...][:, :1]
    eu = cols == uu
    ev = cols == vv
    valid = jnp.logical_and(~eu, ~ev).astype(jnp.float32)
    bu = dc_ref[:B, :] > 0
    bv = dc_ref[B:, :] > 0
    ru = acc[:B, :] > 0
    rv = acc[B:, :] > 0
    d2u = ru & ~bu & ~eu
    d2v = rv & ~bv & ~ev
    infu = ~bu & ~d2u & ~eu
    infv = ~bv & ~d2v & ~ev

    f32 = lambda m: m.astype(jnp.float32)
    t11 = f32(bu & bv)
    t12 = f32(bu & d2v) + f32(d2u & bv)
    t1i = f32(bu & infv) + f32(infu & bv)
    t22 = f32(d2u & d2v)
    t2i = f32(d2u & infv) + f32(infu & d2v)

    col8 = lax.broadcasted_iota(jnp.int32, (CB, 8), 1)
    part = jnp.zeros((B, 8), jnp.float32)
    for m, t in enumerate((t11, t12, t1i, t22, t2i)):
        sel = (col8 == m).astype(jnp.float32)
        part += jnp.dot(t * valid, sel, preferred_element_type=jnp.float32)
    cacc[...] += part

    @pl.when(j == pl.num_programs(0) - 1)
    def _():
        cnt = cacc[...]
        h = jnp.dot(cnt, swt_ref[...], preferred_element_type=jnp.float32)
        h = (h + sb_ref[...]) * bns_ref[...] + bnb_ref[...]
        h = jnp.maximum(h, 0.0)
        xuv = xu_ref[...] * xv_ref[...]
        z = jnp.dot(xuv, w0x_ref[...], preferred_element_type=jnp.float32)
        z += jnp.dot(h, w0h_ref[...], preferred_element_type=jnp.float32)
        z = jnp.maximum(z + b0_ref[...], 0.0)
        z = jnp.dot(z, w1t_ref[...], preferred_element_type=jnp.float32)
        z = jnp.maximum(z + b1_ref[...], 0.0)
        o = jnp.dot(z, w2p_ref[...], preferred_element_type=jnp.float32)
        o_ref[...] = o + b2p_ref[...]


def _fused(d, a, ub, vb, xu, xv, swt8, sb8, bns8, bnb8, w0xt, w0ht8, b0r,
           w1t, b1r, w2p, b2p):
    full = lambda shape: pl.BlockSpec(shape, lambda j: (0,) * len(shape))
    return pl.pallas_call(
        _fused_body,
        grid=(N // CB,),
        in_specs=[
            full((2 * B, N)),
            pl.BlockSpec((N, CB), lambda j: (0, j)),
            pl.BlockSpec((2 * B, CB), lambda j: (0, j)),
            full((B, 128)),
            full((B, 128)),
            full((B, 128)),
            full((B, 128)),
            full((8, 8)),
            full((1, 8)),
            full((1, 8)),
            full((1, 8)),
            full((128, 256)),
            full((8, 256)),
            full((1, 256)),
            full((256, 256)),
            full((1, 256)),
            full((256, 128)),
            full((1, 128)),
        ],
        out_specs=full((B, 128)),
        out_shape=jax.ShapeDtypeStruct((B, 128), jnp.float32),
        scratch_shapes=[pltpu.VMEM((B, 8), jnp.float32)],
        compiler_params=pltpu.CompilerParams(
            dimension_semantics=("arbitrary",)),
    )(d, a, d, ub, vb, xu, xv, swt8, sb8, bns8, bnb8, w0xt, w0ht8, b0r, w1t,
      b1r, w2p, b2p)


def kernel(x, adj, edges, struct_w, struct_b, bn_gamma, bn_beta, w0, b0, w1,
           b1, w2, b2):
    a = _build_a(adj)
    u = edges[0]
    v = edges[1]
    uv = jnp.concatenate([u, v])
    d, xu, xv = _gather(a, x, uv)
    d = d.astype(jnp.bfloat16)

    ub = jnp.broadcast_to(u[:, None], (B, 128))
    vb = jnp.broadcast_to(v[:, None], (B, 128))
    swt8 = jnp.zeros((8, 8), jnp.float32).at[:5, :5].set(struct_w.T)
    sb8 = jnp.zeros((1, 8), jnp.float32).at[0, :5].set(struct_b)
    bns8 = jnp.zeros((1, 8), jnp.float32).at[0, :5].set(
        bn_gamma / np.sqrt(1.0 + 1e-05))
    bnb8 = jnp.zeros((1, 8), jnp.float32).at[0, :5].set(bn_beta)
    w0xt = w0[:, :128].T
    w0ht8 = jnp.zeros((8, 256), jnp.float32).at[:5].set(w0[:, 128:].T)
    b0r = b0[None, :]
    w1t = w1.T
    b1r = b1[None, :]
    w2p = jnp.zeros((256, 128), jnp.float32).at[:, 0].set(w2[0])
    b2p = jnp.zeros((1, 128), jnp.float32).at[0, 0].set(b2[0])

    out = _fused(d, a, ub, vb, xu, xv, swt8, sb8, bns8, bnb8, w0xt, w0ht8,
                 b0r, w1t, b1r, w2p, b2p)
    return out[:, :1]


# speedup vs baseline: 4.1009x; 1.1927x over previous
import jax
import jax.numpy as jnp
import numpy as np
from jax import lax
from jax.experimental import pallas as pl
from jax.experimental.pallas import tpu as pltpu
from jax.experimental.pallas import tpu_sc as plsc

N = 4096
E = 65536
B = 1024

SLAB_AROWS = 256
NUM_SLABS = N // SLAB_AROWS
SLAB_ELEMS = SLAB_AROWS * N
JUNK_ELEMS = 4096
EDGES_PER_TILE = E // 16
ZCHUNK = (SLAB_ELEMS + JUNK_ELEMS) // 16 // 4


def _scatter_body(edges_hbm, a_out, src_v, dst_v, slab, idxb, ones_v, zbuf,
                  sem):
    c = lax.axis_index("c")
    s = lax.axis_index("s")
    iota16 = lax.iota(jnp.int32, 16)
    ones16 = jnp.ones((16,), jnp.float32)
    zeros16 = jnp.zeros((16,), jnp.float32)

    ebase = s * EDGES_PER_TILE
    pltpu.sync_copy(edges_hbm.at[0, pl.ds(ebase, EDGES_PER_TILE)], src_v)
    pltpu.sync_copy(edges_hbm.at[1, pl.ds(ebase, EDGES_PER_TILE)], dst_v)

    def _z1(i, _):
        zbuf[pl.ds(i * 16, 16)] = zeros16
        return 0
    lax.fori_loop(0, ZCHUNK // 16, _z1, 0)

    for i in range(8):
        ones_v[pl.ds(i * 16, 16)] = ones16

    def _slab_step(k, _):
        slab_id = c * (NUM_SLABS // 2) + k
        row_lo = slab_id * SLAB_AROWS
        elem_base = slab_id * SLAB_ELEMS

        for j in range(4):
            pltpu.sync_copy(
                zbuf, slab.at[pl.ds((s * 4 + j) * ZCHUNK, ZCHUNK)])
        plsc.subcore_barrier()

        def _batch(b, _):
            @pl.when(b >= 2)
            def _():
                pltpu.make_async_copy(ones_v, slab.at[idxb.at[0]],
                                      sem).wait()

            def _build(slot):
                for t in range(4):
                    i = b * 4 + t
                    sv = src_v[pl.ds(i * 16, 16)]
                    dv = dst_v[pl.ds(i * 16, 16)]
                    neq = ((sv - dv) | (dv - sv)) >> 31
                    for which in range(2):
                        rs = sv if which == 0 else dv
                        rd = dv if which == 0 else sv
                        pos = t * 32 + which * 16
                        oor = ((rs - row_lo)
                               | (row_lo + SLAB_AROWS - 1 - rs)) >> 31
                        okm = (~oor) & neq
                        junk = SLAB_ELEMS + ((iota16 * 8 + i * 32 + pos)
                                             & (JUNK_ELEMS - 1))
                        idx = (((rs * N + rd - elem_base) & okm)
                               | (junk & (~okm)))
                        idxb[slot, pl.ds(pos, 16)] = idx
                pltpu.async_copy(ones_v, slab.at[idxb.at[slot]], sem,
                                 add=True)

            @pl.when(b % 2 == 0)
            def _():
                _build(0)

            @pl.when(b % 2 == 1)
            def _():
                _build(1)
            return 0

        lax.fori_loop(0, EDGES_PER_TILE // 64, _batch, 0)
        pltpu.make_async_copy(ones_v, slab.at[idxb.at[0]], sem).wait()
        pltpu.make_async_copy(ones_v, slab.at[idxb.at[0]], sem).wait()
        plsc.subcore_barrier()

        for r in range(16):
            pltpu.async_copy(
                slab.at[pl.ds(s * (SLAB_ELEMS // 16) + r * N, N)],
                a_out.at[row_lo + s * 16 + r], sem)
        for r in range(16):
            pltpu.make_async_copy(slab.at[pl.ds(0, N)], a_out.at[row_lo],
                                  sem).wait()
        plsc.subcore_barrier()
        return 0

    lax.fori_loop(0, NUM_SLABS // 2, _slab_step, 0)


def _build_a(adj):
    mesh = plsc.VectorSubcoreMesh(core_axis_name="c", subcore_axis_name="s")
    k = pl.kernel(
        _scatter_body,
        out_type=jax.ShapeDtypeStruct((N, N), jnp.float32),
        mesh=mesh,
        scratch_types=[
            pltpu.VMEM((EDGES_PER_TILE,), jnp.int32),
            pltpu.VMEM((EDGES_PER_TILE,), jnp.int32),
            pltpu.VMEM_SHARED((SLAB_ELEMS + JUNK_ELEMS,), jnp.float32),
            pltpu.VMEM((2, 128), jnp.int32),
            pltpu.VMEM((128,), jnp.float32),
            pltpu.VMEM((ZCHUNK,), jnp.float32),
            pltpu.SemaphoreType.DMA,
        ],
    )
    return k(adj)


def _gather_body(a_hbm, x_hbm, uv_hbm, d_out, xu_out, xv_out, idx_v, rows_a,
                 rows_x, sem):
    c = lax.axis_index("c")
    s = lax.axis_index("s")
    wid = s * 2 + c
    base = wid * 64
    pltpu.sync_copy(uv_hbm.at[pl.ds(base, 64)], idx_v)
    for j in range(4):
        pltpu.async_copy(a_hbm.at[idx_v.at[pl.ds(j * 16, 16)]], rows_a,
                         sem).wait()
        pltpu.sync_copy(rows_a, d_out.at[pl.ds(base + j * 16, 16)])
    xbase = wid * 32
    pltpu.sync_copy(uv_hbm.at[pl.ds(xbase, 32)], idx_v.at[pl.ds(0, 32)])
    pltpu.async_copy(x_hbm.at[idx_v.at[pl.ds(0, 32)]], rows_x, sem).wait()
    pltpu.sync_copy(rows_x, xu_out.at[pl.ds(xbase, 32)])
    pltpu.sync_copy(uv_hbm.at[pl.ds(B + xbase, 32)], idx_v.at[pl.ds(0, 32)])
    pltpu.async_copy(x_hbm.at[idx_v.at[pl.ds(0, 32)]], rows_x, sem).wait()
    pltpu.sync_copy(rows_x, xv_out.at[pl.ds(xbase, 32)])


def _gather(a, x, uv):
    mesh = plsc.VectorSubcoreMesh(core_axis_name="c", subcore_axis_name="s")
    k = pl.kernel(
        _gather_body,
        out_type=(
            jax.ShapeDtypeStruct((2 * B, N), jnp.float32),
            jax.ShapeDtypeStruct((B, 128), jnp.float32),
            jax.ShapeDtypeStruct((B, 128), jnp.float32),
        ),
        mesh=mesh,
        scratch_types=[
            pltpu.VMEM((64,), jnp.int32),
            pltpu.VMEM((16, N), jnp.float32),
            pltpu.VMEM((32, 128), jnp.float32),
            pltpu.SemaphoreType.DMA,
        ],
    )
    return k(a, x, uv)


CB = 512
KB = 512


def _fused_body(d_ref, a_ref, dc_ref, ub_ref, vb_ref, xu_ref, xv_ref,
                swt_ref, sb_ref, bns_ref, bnb_ref, w0x_ref, w0h_ref, b0_ref,
                w1t_ref, b1_ref, w2p_ref, b2p_ref, o_ref, cacc):
    j = pl.program_id(0)

    @pl.when(j == 0)
    def _():
        cacc[...] = jnp.zeros_like(cacc)

    acc = jnp.zeros((2 * B, CB), jnp.float32)
    for k in range(N // KB):
        acc += jnp.dot(
            d_ref[:, k * KB:(k + 1) * KB],
            a_ref[k * KB:(k + 1) * KB, :].astype(jnp.bfloat16),
            preferred_element_type=jnp.float32)

    cols = j * CB + lax.broadcasted_iota(jnp.int32, (B, CB), 1)
    uu = ub_ref[...][:, :1]
    vv = vb_ref[...][:, :1]
    eu = cols == uu
    ev = cols == vv
    valid = jnp.logical_and(~eu, ~ev).astype(jnp.float32)
    bu = dc_ref[:B, :] > 0
    bv = dc_ref[B:, :] > 0
    ru = acc[:B, :] > 0
    rv = acc[B:, :] > 0
    d2u = ru & ~bu & ~eu
    d2v = rv & ~bv & ~ev
    infu = ~bu & ~d2u & ~eu
    infv = ~bv & ~d2v & ~ev

    f32 = lambda m: m.astype(jnp.float32)
    t11 = f32(bu & bv)
    t12 = f32(bu & d2v) + f32(d2u & bv)
    t1i = f32(bu & infv) + f32(infu & bv)
    t22 = f32(d2u & d2v)
    t2i = f32(d2u & infv) + f32(infu & d2v)

    col8 = lax.broadcasted_iota(jnp.int32, (CB, 8), 1)
    part = jnp.zeros((B, 8), jnp.float32)
    for m, t in enumerate((t11, t12, t1i, t22, t2i)):
        sel = (col8 == m).astype(jnp.float32)
        part += jnp.dot(t * valid, sel, preferred_element_type=jnp.float32)
    cacc[...] += part

    @pl.when(j == pl.num_programs(0) - 1)
    def _():
        cnt = cacc[...]
        h = jnp.dot(cnt, swt_ref[...], preferred_element_type=jnp.float32)
        h = (h + sb_ref[...]) * bns_ref[...] + bnb_ref[...]
        h = jnp.maximum(h, 0.0)
        xuv = xu_ref[...] * xv_ref[...]
        z = jnp.dot(xuv, w0x_ref[...], preferred_element_type=jnp.float32)
        z += jnp.dot(h, w0h_ref[...], preferred_element_type=jnp.float32)
        z = jnp.maximum(z + b0_ref[...], 0.0)
        z = jnp.dot(z, w1t_ref[...], preferred_element_type=jnp.float32)
        z = jnp.maximum(z + b1_ref[...], 0.0)
        o = jnp.dot(z, w2p_ref[...], preferred_element_type=jnp.float32)
        o_ref[...] = o + b2p_ref[...]


def _fused(d, a, ub, vb, xu, xv, swt8, sb8, bns8, bnb8, w0xt, w0ht8, b0r,
           w1t, b1r, w2p, b2p):
    full = lambda shape: pl.BlockSpec(shape, lambda j: (0,) * len(shape))
    return pl.pallas_call(
        _fused_body,
        grid=(N // CB,),
        in_specs=[
            full((2 * B, N)),
            pl.BlockSpec((N, CB), lambda j: (0, j)),
            pl.BlockSpec((2 * B, CB), lambda j: (0, j)),
            full((B, 128)),
            full((B, 128)),
            full((B, 128)),
            full((B, 128)),
            full((8, 8)),
            full((1, 8)),
            full((1, 8)),
            full((1, 8)),
            full((128, 256)),
            full((8, 256)),
            full((1, 256)),
            full((256, 256)),
            full((1, 256)),
            full((256, 128)),
            full((1, 128)),
        ],
        out_specs=full((B, 128)),
        out_shape=jax.ShapeDtypeStruct((B, 128), jnp.float32),
        scratch_shapes=[pltpu.VMEM((B, 8), jnp.float32)],
        compiler_params=pltpu.CompilerParams(
            dimension_semantics=("arbitrary",)),
    )(d, a, d, ub, vb, xu, xv, swt8, sb8, bns8, bnb8, w0xt, w0ht8, b0r, w1t,
      b1r, w2p, b2p)


def kernel(x, adj, edges, struct_w, struct_b, bn_gamma, bn_beta, w0, b0, w1,
           b1, w2, b2):
    a = _build_a(adj)
    u = edges[0]
    v = edges[1]
    uv = jnp.concatenate([u, v])
    d, xu, xv = _gather(a, x, uv)
    d = d.astype(jnp.bfloat16)

    ub = jnp.broadcast_to(u[:, None], (B, 128))
    vb = jnp.broadcast_to(v[:, None], (B, 128))
    swt8 = jnp.zeros((8, 8), jnp.float32).at[:5, :5].set(struct_w.T)
    sb8 = jnp.zeros((1, 8), jnp.float32).at[0, :5].set(struct_b)
    bns8 = jnp.zeros((1, 8), jnp.float32).at[0, :5].set(
        bn_gamma / np.sqrt(1.0 + 1e-05))
    bnb8 = jnp.zeros((1, 8), jnp.float32).at[0, :5].set(bn_beta)
    w0xt = w0[:, :128].T
    w0ht8 = jnp.zeros((8, 256), jnp.float32).at[:5].set(w0[:, 128:].T)
    b0r = b0[None, :]
    w1t = w1.T
    b1r = b1[None, :]
    w2p = jnp.zeros((256, 128), jnp.float32).at[:, 0].set(w2[0])
    b2p = jnp.zeros((1, 128), jnp.float32).at[0, 0].set(b2[0])

    out = _fused(d, a, ub, vb, xu, xv, swt8, sb8, bns8, bnb8, w0xt, w0ht8,
                 b0r, w1t, b1r, w2p, b2p)
    return out[:, :1]
